# baseline (speedup 1.0000x reference)
import jax
import jax.numpy as jnp
from jax.experimental import pallas as pl

N = 128
R = 7
O = 32
BB = 16


def _gcn_kernel(x_ref, wrel_ref, wroot_ref, bias_ref, out_ref):
    wrel = wrel_ref[...]
    wroot = wroot_ref[...]
    bias = bias_ref[...]
    ones = jnp.ones((R, 1, N), dtype=jnp.bfloat16)

    blks = [x_ref[bb] for bb in range(BB)]
    degs = [jax.lax.dot_general(ones, blk, (((2,), (1,)), ((0,), (0,))),
                                preferred_element_type=jnp.float32)
            for blk in blks]
    recips = [(1.0 / jnp.maximum(d, 1.0)).astype(jnp.bfloat16)
              for d in degs]
    ms = [jax.lax.dot_general(blk, wrel, (((2,), (1,)), ((0,), (0,))),
                              preferred_element_type=jnp.float32)
          for blk in blks]
    ats = [(blk * rc).reshape(R * N, N) for blk, rc in zip(blks, recips)]
    out_rels = [jax.lax.dot_general(at, m.astype(jnp.bfloat16).reshape(R * N, O),
                                    (((0,), (0,)), ((), ())),
                                    preferred_element_type=jnp.float32)
                for at, m in zip(ats, ms)]
    hroots = [jnp.sum(blk, axis=0) for blk in blks]
    roots = [jax.lax.dot_general(h, wroot, (((1,), (0,)), ((), ())),
                                 preferred_element_type=jnp.float32)
             for h in hroots]
    for bb in range(BB):
        out_ref[bb] = out_rels[bb] + roots[bb] * (1.0 / R) + bias


@jax.jit
def kernel(x, W_rel, W_root, bias):
    B = x.shape[0]
    xt = jnp.transpose(x, (0, 3, 1, 2)).astype(jnp.bfloat16)
    bias2 = bias.reshape(1, O)
    return pl.pallas_call(
        _gcn_kernel,
        grid=(B // BB,),
        in_specs=[
            pl.BlockSpec((BB, R, N, N), lambda b: (b, 0, 0, 0)),
            pl.BlockSpec((R, N, O), lambda b: (0, 0, 0)),
            pl.BlockSpec((N, O), lambda b: (0, 0)),
            pl.BlockSpec((1, O), lambda b: (0, 0)),
        ],
        out_specs=pl.BlockSpec((BB, N, O), lambda b: (b, 0, 0)),
        out_shape=jax.ShapeDtypeStruct((B, N, O), jnp.float32),
    )(xt, W_rel.astype(jnp.bfloat16), W_root.astype(jnp.bfloat16), bias2)


# speedup vs baseline: 1.8006x; 1.8006x over previous
import jax
import jax.numpy as jnp
from jax.experimental import pallas as pl

N = 128
R = 7
O = 32
BB = 16


def _gcn_kernel(x_ref, wrel_ref, wroot_ref, bias_ref, out_ref):
    wrel = wrel_ref[...]
    wroot = wroot_ref[...]
    bias = bias_ref[...]

    blks = [x_ref[bb].astype(jnp.bfloat16) for bb in range(BB)]
    degs = [jnp.sum(x_ref[bb], axis=1, keepdims=True) for bb in range(BB)]
    recips = [(1.0 / jnp.maximum(d, 1.0)).astype(jnp.bfloat16)
              for d in degs]
    ms = [jax.lax.dot_general(blk, wrel, (((2,), (1,)), ((0,), (0,))),
                              preferred_element_type=jnp.float32)
          for blk in blks]
    ats = [(blk * rc).reshape(R * N, N) for blk, rc in zip(blks, recips)]
    out_rels = [jax.lax.dot_general(at, m.astype(jnp.bfloat16).reshape(R * N, O),
                                    (((0,), (0,)), ((), ())),
                                    preferred_element_type=jnp.float32)
                for at, m in zip(ats, ms)]
    hroots = [jnp.sum(blk, axis=0) for blk in blks]
    roots = [jax.lax.dot_general(h, wroot, (((1,), (0,)), ((), ())),
                                 preferred_element_type=jnp.float32)
             for h in hroots]
    for bb in range(BB):
        out_ref[bb] = out_rels[bb] + roots[bb] * (1.0 / R) + bias


@jax.jit
def kernel(x, W_rel, W_root, bias):
    B = x.shape[0]
    xt = jnp.transpose(x, (0, 3, 1, 2))
    bias2 = bias.reshape(1, O)
    return pl.pallas_call(
        _gcn_kernel,
        grid=(B // BB,),
        in_specs=[
            pl.BlockSpec((BB, R, N, N), lambda b: (b, 0, 0, 0)),
            pl.BlockSpec((R, N, O), lambda b: (0, 0, 0)),
            pl.BlockSpec((N, O), lambda b: (0, 0)),
            pl.BlockSpec((1, O), lambda b: (0, 0)),
        ],
        out_specs=pl.BlockSpec((BB, N, O), lambda b: (b, 0, 0)),
        out_shape=jax.ShapeDtypeStruct((B, N, O), jnp.float32),
    )(xt, W_rel.astype(jnp.bfloat16), W_root.astype(jnp.bfloat16), bias2)
